# Initial kernel scaffold; baseline (speedup 1.0000x reference)
import jax
import jax.numpy as jnp
from jax.experimental import pallas as pl


def kernel(x, edge_index, batch, W1, b1, W2, b2, gate_W, gate_b, reg_W, reg_b):
    raise NotImplementedError("write your pallas kernel here")


# speedup vs baseline: 28.0813x; 28.0813x over previous
import functools

import jax
import jax.numpy as jnp
from jax import lax
from jax.experimental import pallas as pl
from jax.experimental.pallas import tpu as pltpu
from jax.experimental.pallas import tpu_sc as plsc

N = 100000
E = 1600000
B = 64
NC = 2
NS = 16
K = 1000
NPAD = 100096
ROWS_PER_TILE = NPAD // NS
BS = 2000
NB = N // BS


def _sc_mesh():
  return plsc.VectorSubcoreMesh(
      core_axis_name="c", subcore_axis_name="s", num_cores=NC, num_subcores=NS)


def _zero_acc(zrow_hbm, acc, tid):
  pltpu.sync_copy(zrow_hbm, acc.at[pl.ds(tid * ROWS_PER_TILE, ROWS_PER_TILE)])


def _writeout(acc, out_hbm, slab, tid):
  pltpu.sync_copy(
      acc.at[pl.ds(tid * ROWS_PER_TILE, ROWS_PER_TILE)],
      out_hbm.at[slab, pl.ds(tid * ROWS_PER_TILE, ROWS_PER_TILE)])


def _deg_body(dst_hbm, ones_hbm, zrow_hbm, out_hbm, didx, rows, acc, *, slab):
  tid = lax.axis_index("s")
  _zero_acc(zrow_hbm, acc, tid)
  pltpu.sync_copy(ones_hbm, rows)
  plsc.subcore_barrier()
  e_per_tile = E // (NC * NS)
  base = slab * (E // NC) + tid * e_per_tile

  def chunk(i, carry):
    pltpu.sync_copy(dst_hbm.at[pl.ds(base + i * K, K)], didx)
    pltpu.sync_copy(rows, acc.at[didx], add=True)
    return carry

  lax.fori_loop(0, e_per_tile // K, chunk, 0)
  plsc.subcore_barrier()
  _writeout(acc, out_hbm, slab, tid)


def _deg_call(dst, ones_rows, zrow):
  @functools.partial(
      pl.kernel,
      out_type=jax.ShapeDtypeStruct((NC, NPAD, 16), jnp.float32),
      mesh=_sc_mesh(),
      compiler_params=pltpu.CompilerParams(use_tc_tiling_on_sc=False),
      scratch_types=[
          pltpu.VMEM((K,), jnp.int32),
          pltpu.VMEM((K, 16), jnp.float32),
          pltpu.VMEM_SHARED((NPAD, 16), jnp.float32),
      ],
  )
  def deg_kernel(dst_hbm, ones_hbm, zrow_hbm, out_hbm, didx, rows, acc):
    c = lax.axis_index("c")

    @pl.when(c == 0)
    def _():
      _deg_body(dst_hbm, ones_hbm, zrow_hbm, out_hbm, didx, rows, acc, slab=0)

    @pl.when(c == 1)
    def _():
      _deg_body(dst_hbm, ones_hbm, zrow_hbm, out_hbm, didx, rows, acc, slab=1)

  return deg_kernel(dst, ones_rows, zrow)


def _agg_body(table_hbm, src_hbm, dst_hbm, zrow_hbm, out_hbm, sidx, didx, rows,
              gsem, acc, *, slab, e_lo, e_per_tile):
  tid = lax.axis_index("s")
  _zero_acc(zrow_hbm, acc, tid)
  plsc.subcore_barrier()
  base = e_lo + tid * e_per_tile

  def chunk(i, carry):
    off = base + i * K
    pltpu.sync_copy(src_hbm.at[pl.ds(off, K)], sidx)
    pltpu.sync_copy(dst_hbm.at[pl.ds(off, K)], didx)
    pltpu.async_copy(table_hbm.at[sidx], rows, gsem).wait()
    pltpu.sync_copy(rows, acc.at[didx], add=True)
    return carry

  lax.fori_loop(0, e_per_tile // K, chunk, 0)
  plsc.subcore_barrier()
  _writeout(acc, out_hbm, slab, tid)


def _agg_scratch():
  return [
      pltpu.VMEM((K,), jnp.int32),
      pltpu.VMEM((K,), jnp.int32),
      pltpu.VMEM((K, 16), jnp.float32),
      pltpu.SemaphoreType.DMA,
      pltpu.VMEM_SHARED((NPAD, 16), jnp.float32),
  ]


def _agg1_call(xn, src, dst, zrow):
  @functools.partial(
      pl.kernel,
      out_type=jax.ShapeDtypeStruct((NC, NPAD, 16), jnp.float32),
      mesh=_sc_mesh(),
      compiler_params=pltpu.CompilerParams(use_tc_tiling_on_sc=False),
      scratch_types=_agg_scratch(),
  )
  def agg1_kernel(table_hbm, src_hbm, dst_hbm, zrow_hbm, out_hbm, sidx, didx,
                  rows, gsem, acc):
    c = lax.axis_index("c")
    e_per_tile = E // (NC * NS)

    @pl.when(c == 0)
    def _():
      _agg_body(table_hbm, src_hbm, dst_hbm, zrow_hbm, out_hbm, sidx, didx,
                rows, gsem, acc, slab=0, e_lo=0, e_per_tile=e_per_tile)

    @pl.when(c == 1)
    def _():
      _agg_body(table_hbm, src_hbm, dst_hbm, zrow_hbm, out_hbm, sidx, didx,
                rows, gsem, acc, slab=1, e_lo=E // 2, e_per_tile=e_per_tile)

  return agg1_kernel(xn, src, dst, zrow)


def _agg2_call(g1a, g1b, src, dst, zrow):
  @functools.partial(
      pl.kernel,
      out_type=jax.ShapeDtypeStruct((NC, NPAD, 16), jnp.float32),
      mesh=_sc_mesh(),
      compiler_params=pltpu.CompilerParams(use_tc_tiling_on_sc=False),
      scratch_types=_agg_scratch(),
  )
  def agg2_kernel(ta_hbm, tb_hbm, src_hbm, dst_hbm, zrow_hbm, out_hbm, sidx,
                  didx, rows, gsem, acc):
    c = lax.axis_index("c")
    e_per_tile = E // NS

    @pl.when(c == 0)
    def _():
      _agg_body(ta_hbm, src_hbm, dst_hbm, zrow_hbm, out_hbm, sidx, didx,
                rows, gsem, acc, slab=0, e_lo=0, e_per_tile=e_per_tile)

    @pl.when(c == 1)
    def _():
      _agg_body(tb_hbm, src_hbm, dst_hbm, zrow_hbm, out_hbm, sidx, didx,
                rows, gsem, acc, slab=1, e_lo=0, e_per_tile=e_per_tile)

  return agg2_kernel(g1a, g1b, src, dst, zrow)


def _prep_kernel(c0, c1, xp, xn_out, dis_out):
  deg = 1.0 + c0[0][:, 0:1] + c1[0][:, 0:1]
  dis = lax.rsqrt(deg)
  dis_out[...] = dis
  xn_out[...] = xp[...] * dis


def _prep_call(degcnt, xp):
  return pl.pallas_call(
      _prep_kernel,
      grid=(NB,),
      in_specs=[
          pl.BlockSpec((1, BS, 16), lambda i: (0, i, 0)),
          pl.BlockSpec((1, BS, 16), lambda i: (1, i, 0)),
          pl.BlockSpec((BS, 16), lambda i: (i, 0)),
      ],
      out_specs=[
          pl.BlockSpec((BS, 16), lambda i: (i, 0)),
          pl.BlockSpec((BS, 1), lambda i: (i, 0)),
      ],
      out_shape=[
          jax.ShapeDtypeStruct((N, 16), jnp.float32),
          jax.ShapeDtypeStruct((N, 1), jnp.float32),
      ],
  )(degcnt, degcnt, xp)


def _mid_kernel(p0, p1, xn, dis, w1, b1, g1a_out, g1b_out):
  d = dis[...]
  z1 = (p0[0] + p1[0] + xn[...]) * d
  h1 = jnp.dot(z1, w1[...], preferred_element_type=jnp.float32,
               precision=lax.Precision.HIGHEST) + b1[...]
  g1 = jnp.maximum(h1, 0.0) * d
  g1a_out[...] = g1[:, :16]
  g1b_out[...] = g1[:, 16:]


def _mid_call(p, xn, dis, w1p, b1):
  return pl.pallas_call(
      _mid_kernel,
      grid=(NB,),
      in_specs=[
          pl.BlockSpec((1, BS, 16), lambda i: (0, i, 0)),
          pl.BlockSpec((1, BS, 16), lambda i: (1, i, 0)),
          pl.BlockSpec((BS, 16), lambda i: (i, 0)),
          pl.BlockSpec((BS, 1), lambda i: (i, 0)),
          pl.BlockSpec((16, 32), lambda i: (0, 0)),
          pl.BlockSpec((1, 32), lambda i: (0, 0)),
      ],
      out_specs=[
          pl.BlockSpec((BS, 16), lambda i: (i, 0)),
          pl.BlockSpec((BS, 16), lambda i: (i, 0)),
      ],
      out_shape=[
          jax.ShapeDtypeStruct((N, 16), jnp.float32),
          jax.ShapeDtypeStruct((N, 16), jnp.float32),
      ],
  )(p, p, xn, dis, w1p, b1)


def _pool_kernel(q0, q1, g1a, g1b, dis, bt, w2a, w2b, b2, gw, gb, rw, rb,
                 out, m_s, den_s, numt_s):
  i = pl.program_id(0)

  @pl.when(i == 0)
  def _():
    m_s[...] = jnp.full((1, B), -1e30, jnp.float32)
    den_s[...] = jnp.zeros((1, B), jnp.float32)
    numt_s[...] = jnp.zeros((64, B), jnp.float32)

  d = dis[...]
  z2a = (q0[0] + g1a[...]) * d
  z2b = (q1[0] + g1b[...]) * d
  h2 = (jnp.dot(z2a, w2a[...], preferred_element_type=jnp.float32,
                precision=lax.Precision.HIGHEST)
        + jnp.dot(z2b, w2b[...], preferred_element_type=jnp.float32,
                  precision=lax.Precision.HIGHEST) + b2[...])
  gate = jnp.dot(h2, gw[...], preferred_element_type=jnp.float32,
                 precision=lax.Precision.HIGHEST) + gb[...]
  seg = bt[...]
  onehot = seg == lax.broadcasted_iota(jnp.int32, (1, B), 1)
  g_masked = jnp.where(onehot, gate, -1e30)
  bmax = jnp.max(g_masked, axis=0, keepdims=True)
  mold = m_s[...]
  mnew = jnp.maximum(mold, bmax)
  m_s[...] = mnew
  r = jnp.exp(mold - mnew)
  e_term = jnp.where(onehot, jnp.exp(g_masked - mnew), 0.0)
  den_s[...] = den_s[...] * r + jnp.sum(e_term, axis=0, keepdims=True)
  numt_s[...] = numt_s[...] * r + lax.dot_general(
      h2, e_term, (((0,), (0,)), ((), ())),
      preferred_element_type=jnp.float32, precision=lax.Precision.HIGHEST)

  @pl.when(i == NB - 1)
  def _():
    pooled_t = numt_s[...] / (den_s[...] + 1e-16)
    out[...] = lax.dot_general(
        pooled_t, rw[...], (((0,), (0,)), ((), ())),
        preferred_element_type=jnp.float32,
        precision=lax.Precision.HIGHEST) + rb[...]


def _pool_call(q, g1a, g1b, dis, bt, w2a, w2b, b2, gw, gb, rw, rb):
  return pl.pallas_call(
      _pool_kernel,
      grid=(NB,),
      in_specs=[
          pl.BlockSpec((1, BS, 16), lambda i: (0, i, 0)),
          pl.BlockSpec((1, BS, 16), lambda i: (1, i, 0)),
          pl.BlockSpec((BS, 16), lambda i: (i, 0)),
          pl.BlockSpec((BS, 16), lambda i: (i, 0)),
          pl.BlockSpec((BS, 1), lambda i: (i, 0)),
          pl.BlockSpec((BS, 1), lambda i: (i, 0)),
          pl.BlockSpec((16, 64), lambda i: (0, 0)),
          pl.BlockSpec((16, 64), lambda i: (0, 0)),
          pl.BlockSpec((1, 64), lambda i: (0, 0)),
          pl.BlockSpec((64, 1), lambda i: (0, 0)),
          pl.BlockSpec((1, 1), lambda i: (0, 0)),
          pl.BlockSpec((64, 3), lambda i: (0, 0)),
          pl.BlockSpec((1, 3), lambda i: (0, 0)),
      ],
      out_specs=pl.BlockSpec((B, 3), lambda i: (0, 0)),
      out_shape=jax.ShapeDtypeStruct((B, 3), jnp.float32),
      scratch_shapes=[
          pltpu.VMEM((1, B), jnp.float32),
          pltpu.VMEM((1, B), jnp.float32),
          pltpu.VMEM((64, B), jnp.float32),
      ],
  )(q, q, g1a, g1b, dis, bt, w2a, w2b, b2, gw, gb, rw, rb)


def kernel(x, edge_index, batch, W1, b1, W2, b2, gate_W, gate_b, reg_W, reg_b):
  src = edge_index[0]
  dst = edge_index[1]
  zrow = jnp.zeros((ROWS_PER_TILE, 16), jnp.float32)
  ones_rows = jnp.zeros((K, 16), jnp.float32).at[:, 0].set(1.0)
  xp = jnp.pad(x, ((0, 0), (0, 16 - x.shape[1])))
  w1p = jnp.pad(W1, ((0, 16 - W1.shape[0]), (0, 0)))

  degcnt = _deg_call(dst, ones_rows, zrow)
  xn, dis = _prep_call(degcnt, xp)
  p = _agg1_call(xn, src, dst, zrow)
  g1a, g1b = _mid_call(p, xn, dis, w1p, b1.reshape(1, 32))
  q = _agg2_call(g1a, g1b, src, dst, zrow)
  out = _pool_call(q, g1a, g1b, dis, batch.reshape(N, 1), W2[:16], W2[16:],
                   b2.reshape(1, 64), gate_W, gate_b.reshape(1, 1), reg_W,
                   reg_b.reshape(1, 3))
  return out


# speedup vs baseline: 31.4417x; 1.1197x over previous
import functools

import jax
import jax.numpy as jnp
from jax import lax
from jax.experimental import pallas as pl
from jax.experimental.pallas import tpu as pltpu
from jax.experimental.pallas import tpu_sc as plsc

N = 100000
E = 1600000
B = 64
NC = 2
NS = 16
K = 1000
K2 = 800
NPAD = 100096
ROWS_PER_TILE = NPAD // NS
BS = 4000
NB = N // BS


def _sc_mesh():
  return plsc.VectorSubcoreMesh(
      core_axis_name="c", subcore_axis_name="s", num_cores=NC, num_subcores=NS)


def _zero_acc(zrow_hbm, acc, tid):
  pltpu.sync_copy(zrow_hbm, acc.at[pl.ds(tid * ROWS_PER_TILE, ROWS_PER_TILE)])


def _writeout(acc, out_hbm, slab, tid):
  pltpu.sync_copy(
      acc.at[pl.ds(tid * ROWS_PER_TILE, ROWS_PER_TILE)],
      out_hbm.at[slab, pl.ds(tid * ROWS_PER_TILE, ROWS_PER_TILE)])


def _deg_body(ei_hbm, ones_hbm, zrow_hbm, out_hbm, didx, rows, acc, *, slab):
  tid = lax.axis_index("s")
  _zero_acc(zrow_hbm, acc, tid)
  pltpu.sync_copy(ones_hbm, rows)
  plsc.subcore_barrier()
  e_per_tile = E // (NC * NS)
  base = slab * (E // NC) + tid * e_per_tile

  def chunk(i, carry):
    pltpu.sync_copy(ei_hbm.at[1, pl.ds(base + i * K, K)], didx)
    pltpu.sync_copy(rows, acc.at[didx], add=True)
    return carry

  lax.fori_loop(0, e_per_tile // K, chunk, 0)
  plsc.subcore_barrier()
  _writeout(acc, out_hbm, slab, tid)


def _deg_call(ei, ones_rows, zrow):
  @functools.partial(
      pl.kernel,
      out_type=jax.ShapeDtypeStruct((NC, NPAD, 16), jnp.float32),
      mesh=_sc_mesh(),
      compiler_params=pltpu.CompilerParams(use_tc_tiling_on_sc=False),
      scratch_types=[
          pltpu.VMEM((K,), jnp.int32),
          pltpu.VMEM((K, 16), jnp.float32),
          pltpu.VMEM_SHARED((NPAD, 16), jnp.float32),
      ],
  )
  def deg_kernel(ei_hbm, ones_hbm, zrow_hbm, out_hbm, didx, rows, acc):
    c = lax.axis_index("c")

    @pl.when(c == 0)
    def _():
      _deg_body(ei_hbm, ones_hbm, zrow_hbm, out_hbm, didx, rows, acc, slab=0)

    @pl.when(c == 1)
    def _():
      _deg_body(ei_hbm, ones_hbm, zrow_hbm, out_hbm, didx, rows, acc, slab=1)

  return deg_kernel(ei, ones_rows, zrow)


def _agg1_body(table_hbm, ei_hbm, zrow_hbm, out_hbm, sidx, didx, rows, gsem,
               acc, *, slab):
  tid = lax.axis_index("s")
  _zero_acc(zrow_hbm, acc, tid)
  plsc.subcore_barrier()
  e_per_tile = E // (NC * NS)
  base = slab * (E // NC) + tid * e_per_tile

  def chunk(i, carry):
    off = base + i * K
    pltpu.sync_copy(ei_hbm.at[0, pl.ds(off, K)], sidx)
    pltpu.sync_copy(ei_hbm.at[1, pl.ds(off, K)], didx)
    pltpu.async_copy(table_hbm.at[sidx], rows, gsem).wait()
    pltpu.sync_copy(rows, acc.at[didx], add=True)
    return carry

  lax.fori_loop(0, e_per_tile // K, chunk, 0)
  plsc.subcore_barrier()
  _writeout(acc, out_hbm, slab, tid)


def _agg1_call(xn, ei, zrow):
  @functools.partial(
      pl.kernel,
      out_type=jax.ShapeDtypeStruct((NC, NPAD, 16), jnp.float32),
      mesh=_sc_mesh(),
      compiler_params=pltpu.CompilerParams(use_tc_tiling_on_sc=False),
      scratch_types=[
          pltpu.VMEM((K,), jnp.int32),
          pltpu.VMEM((K,), jnp.int32),
          pltpu.VMEM((K, 16), jnp.float32),
          pltpu.SemaphoreType.DMA,
          pltpu.VMEM_SHARED((NPAD, 16), jnp.float32),
      ],
  )
  def agg1_kernel(table_hbm, ei_hbm, zrow_hbm, out_hbm, sidx, didx, rows,
                  gsem, acc):
    c = lax.axis_index("c")

    @pl.when(c == 0)
    def _():
      _agg1_body(table_hbm, ei_hbm, zrow_hbm, out_hbm, sidx, didx, rows,
                 gsem, acc, slab=0)

    @pl.when(c == 1)
    def _():
      _agg1_body(table_hbm, ei_hbm, zrow_hbm, out_hbm, sidx, didx, rows,
                 gsem, acc, slab=1)

  return agg1_kernel(xn, ei, zrow)


def _agg2_body(table_hbm, ei_hbm, zrow_hbm, out_hbm, sa, da, ra, sema,
               sb, db, rb, semb, acc, *, slab):
  tid = lax.axis_index("s")
  _zero_acc(zrow_hbm, acc, tid)
  plsc.subcore_barrier()
  e_per_tile = E // NS
  nch = e_per_tile // K2
  base = tid * e_per_tile

  def load_idx(c, s_ref, d_ref):
    off = base + c * K2
    pltpu.sync_copy(ei_hbm.at[0, pl.ds(off, K2)], s_ref)
    pltpu.sync_copy(ei_hbm.at[1, pl.ds(off, K2)], d_ref)

  load_idx(0, sa, da)
  pltpu.async_copy(table_hbm.at[sa], ra, sema)

  def pair(g, carry):
    c1 = 2 * g + 1

    @pl.when(c1 < nch)
    def _():
      load_idx(c1, sb, db)
      pltpu.async_copy(table_hbm.at[sb], rb, semb)

    pltpu.make_async_copy(table_hbm.at[sa], ra, sema).wait()
    pltpu.sync_copy(ra, acc.at[da], add=True)

    @pl.when(c1 + 1 < nch)
    def _():
      load_idx(c1 + 1, sa, da)
      pltpu.async_copy(table_hbm.at[sa], ra, sema)

    @pl.when(c1 < nch)
    def _():
      pltpu.make_async_copy(table_hbm.at[sb], rb, semb).wait()
      pltpu.sync_copy(rb, acc.at[db], add=True)

    return carry

  lax.fori_loop(0, (nch + 1) // 2, pair, 0)
  plsc.subcore_barrier()
  _writeout(acc, out_hbm, slab, tid)


def _agg2_call(g1a, g1b, ei, zrow):
  @functools.partial(
      pl.kernel,
      out_type=jax.ShapeDtypeStruct((NC, NPAD, 16), jnp.float32),
      mesh=_sc_mesh(),
      compiler_params=pltpu.CompilerParams(use_tc_tiling_on_sc=False),
      scratch_types=[
          pltpu.VMEM((K2,), jnp.int32),
          pltpu.VMEM((K2,), jnp.int32),
          pltpu.VMEM((K2, 16), jnp.float32),
          pltpu.SemaphoreType.DMA,
          pltpu.VMEM((K2,), jnp.int32),
          pltpu.VMEM((K2,), jnp.int32),
          pltpu.VMEM((K2, 16), jnp.float32),
          pltpu.SemaphoreType.DMA,
          pltpu.VMEM_SHARED((NPAD, 16), jnp.float32),
      ],
  )
  def agg2_kernel(ta_hbm, tb_hbm, ei_hbm, zrow_hbm, out_hbm, sa, da, ra,
                  sema, sb, db, rb, semb, acc):
    c = lax.axis_index("c")

    @pl.when(c == 0)
    def _():
      _agg2_body(ta_hbm, ei_hbm, zrow_hbm, out_hbm, sa, da, ra, sema,
                 sb, db, rb, semb, acc, slab=0)

    @pl.when(c == 1)
    def _():
      _agg2_body(tb_hbm, ei_hbm, zrow_hbm, out_hbm, sa, da, ra, sema,
                 sb, db, rb, semb, acc, slab=1)

  return agg2_kernel(g1a, g1b, ei, zrow)


def _prep_kernel(c0, c1, xin, xn_out, dis_out):
  deg = 1.0 + c0[0][:, 0:1] + c1[0][:, 0:1]
  dis = lax.rsqrt(deg)
  dis_out[...] = dis
  xn_out[...] = jnp.concatenate(
      [xin[...] * dis, jnp.zeros((BS, 7), jnp.float32)], axis=1)


def _prep_call(degcnt, x):
  return pl.pallas_call(
      _prep_kernel,
      grid=(NB,),
      in_specs=[
          pl.BlockSpec((1, BS, 16), lambda i: (0, i, 0)),
          pl.BlockSpec((1, BS, 16), lambda i: (1, i, 0)),
          pl.BlockSpec((BS, 9), lambda i: (i, 0)),
      ],
      out_specs=[
          pl.BlockSpec((BS, 16), lambda i: (i, 0)),
          pl.BlockSpec((BS, 1), lambda i: (i, 0)),
      ],
      out_shape=[
          jax.ShapeDtypeStruct((N, 16), jnp.float32),
          jax.ShapeDtypeStruct((N, 1), jnp.float32),
      ],
  )(degcnt, degcnt, x)


def _mid_kernel(p0, p1, xn, dis, w1, b1, g1a_out, g1b_out):
  d = dis[...]
  z1 = (p0[0] + p1[0] + xn[...]) * d
  h1 = jnp.dot(z1[:, :9], w1[...], preferred_element_type=jnp.float32,
               precision=lax.Precision.HIGHEST) + b1[...]
  g1 = jnp.maximum(h1, 0.0) * d
  g1a_out[...] = g1[:, :16]
  g1b_out[...] = g1[:, 16:]


def _mid_call(p, xn, dis, w1, b1):
  return pl.pallas_call(
      _mid_kernel,
      grid=(NB,),
      in_specs=[
          pl.BlockSpec((1, BS, 16), lambda i: (0, i, 0)),
          pl.BlockSpec((1, BS, 16), lambda i: (1, i, 0)),
          pl.BlockSpec((BS, 16), lambda i: (i, 0)),
          pl.BlockSpec((BS, 1), lambda i: (i, 0)),
          pl.BlockSpec((9, 32), lambda i: (0, 0)),
          pl.BlockSpec((1, 32), lambda i: (0, 0)),
      ],
      out_specs=[
          pl.BlockSpec((BS, 16), lambda i: (i, 0)),
          pl.BlockSpec((BS, 16), lambda i: (i, 0)),
      ],
      out_shape=[
          jax.ShapeDtypeStruct((N, 16), jnp.float32),
          jax.ShapeDtypeStruct((N, 16), jnp.float32),
      ],
  )(p, p, xn, dis, w1, b1)


def _pool_kernel(q0, q1, g1a, g1b, dis, bt, w2a, w2b, b2, gw, gb, rw, rb,
                 out, m_s, den_s, numt_s):
  i = pl.program_id(0)

  @pl.when(i == 0)
  def _():
    m_s[...] = jnp.full((1, B), -1e30, jnp.float32)
    den_s[...] = jnp.zeros((1, B), jnp.float32)
    numt_s[...] = jnp.zeros((64, B), jnp.float32)

  d = dis[...]
  z2a = (q0[0] + g1a[...]) * d
  z2b = (q1[0] + g1b[...]) * d
  h2 = (jnp.dot(z2a, w2a[...], preferred_element_type=jnp.float32,
                precision=lax.Precision.HIGHEST)
        + jnp.dot(z2b, w2b[...], preferred_element_type=jnp.float32,
                  precision=lax.Precision.HIGHEST) + b2[...])
  gate = jnp.dot(h2, gw[...], preferred_element_type=jnp.float32,
                 precision=lax.Precision.HIGHEST) + gb[...]
  onehot = bt[...] == lax.broadcasted_iota(jnp.int32, (1, B), 1)
  oh_f = onehot.astype(jnp.float32)
  g_masked = jnp.where(onehot, gate, -1e30)
  bmax = jnp.max(g_masked, axis=0, keepdims=True)
  mold = m_s[...]
  mnew = jnp.maximum(mold, bmax)
  m_s[...] = mnew
  r = jnp.exp(mold - mnew)
  m_row = lax.dot_general(oh_f, mnew, (((1,), (1,)), ((), ())),
                          preferred_element_type=jnp.float32,
                          precision=lax.Precision.HIGHEST)
  a = jnp.exp(gate - m_row)
  den_s[...] = den_s[...] * r + lax.dot_general(
      a, oh_f, (((0,), (0,)), ((), ())),
      preferred_element_type=jnp.float32, precision=lax.Precision.HIGHEST)
  numt_s[...] = numt_s[...] * r + lax.dot_general(
      h2 * a, oh_f, (((0,), (0,)), ((), ())),
      preferred_element_type=jnp.float32, precision=lax.Precision.HIGHEST)

  @pl.when(i == NB - 1)
  def _():
    pooled_t = numt_s[...] / (den_s[...] + 1e-16)
    out[...] = lax.dot_general(
        pooled_t, rw[...], (((0,), (0,)), ((), ())),
        preferred_element_type=jnp.float32,
        precision=lax.Precision.HIGHEST) + rb[...]


def _pool_call(q, g1a, g1b, dis, bt, w2a, w2b, b2, gw, gb, rw, rb):
  return pl.pallas_call(
      _pool_kernel,
      grid=(NB,),
      in_specs=[
          pl.BlockSpec((1, BS, 16), lambda i: (0, i, 0)),
          pl.BlockSpec((1, BS, 16), lambda i: (1, i, 0)),
          pl.BlockSpec((BS, 16), lambda i: (i, 0)),
          pl.BlockSpec((BS, 16), lambda i: (i, 0)),
          pl.BlockSpec((BS, 1), lambda i: (i, 0)),
          pl.BlockSpec((BS, 1), lambda i: (i, 0)),
          pl.BlockSpec((16, 64), lambda i: (0, 0)),
          pl.BlockSpec((16, 64), lambda i: (0, 0)),
          pl.BlockSpec((1, 64), lambda i: (0, 0)),
          pl.BlockSpec((64, 1), lambda i: (0, 0)),
          pl.BlockSpec((1, 1), lambda i: (0, 0)),
          pl.BlockSpec((64, 3), lambda i: (0, 0)),
          pl.BlockSpec((1, 3), lambda i: (0, 0)),
      ],
      out_specs=pl.BlockSpec((B, 3), lambda i: (0, 0)),
      out_shape=jax.ShapeDtypeStruct((B, 3), jnp.float32),
      scratch_shapes=[
          pltpu.VMEM((1, B), jnp.float32),
          pltpu.VMEM((1, B), jnp.float32),
          pltpu.VMEM((64, B), jnp.float32),
      ],
  )(q, q, g1a, g1b, dis, bt, w2a, w2b, b2, gw, gb, rw, rb)


def kernel(x, edge_index, batch, W1, b1, W2, b2, gate_W, gate_b, reg_W, reg_b):
  zrow = jnp.zeros((ROWS_PER_TILE, 16), jnp.float32)
  ones_rows = jnp.zeros((K, 16), jnp.float32).at[:, 0].set(1.0)

  degcnt = _deg_call(edge_index, ones_rows, zrow)
  xn, dis = _prep_call(degcnt, x)
  p = _agg1_call(xn, edge_index, zrow)
  g1a, g1b = _mid_call(p, xn, dis, W1, b1.reshape(1, 32))
  q = _agg2_call(g1a, g1b, edge_index, zrow)
  out = _pool_call(q, g1a, g1b, dis, batch.reshape(N, 1), W2[:16], W2[16:],
                   b2.reshape(1, 64), gate_W, gate_b.reshape(1, 1), reg_W,
                   reg_b.reshape(1, 3))
  return out
